# baseline (speedup 1.0000x reference)
import functools

import jax
import jax.numpy as jnp
from jax import lax
from jax.experimental import pallas as pl
from jax.experimental.pallas import tpu as pltpu
from jax.experimental.pallas import tpu_sc as plsc

_INTERPRET = False


def _topk_body(tc_ref, tr_ref, out_ref, *, T, B, R, N, half):
    i = pl.program_id(0)
    row_base = i * R
    trow = tc_ref[...]
    tall = tr_ref[...]
    diff = jnp.abs(trow - tall)
    col_iota = jax.lax.broadcasted_iota(jnp.int32, (R, N), 1)
    row_iota = jax.lax.broadcasted_iota(jnp.int32, (R, N), 0) + row_base
    same = (row_iota // T) == (col_iota // T)
    masked = jnp.where(same, jnp.float32(1e6), diff)
    key = jax.lax.bitcast_convert_type(
        (jax.lax.bitcast_convert_type(masked, jnp.int32) & ~0x1FFF)
        + (0x00800000 + col_iota),
        jnp.float32)
    cols = []
    if half:
        key = jnp.minimum(key[:, :N // 2], key[:, N // 2:])
        for _ in range(B):
            m = jnp.min(key, axis=1, keepdims=True)
            w = jax.lax.bitcast_convert_type(m, jnp.int32) & 0x1FFF
            cols.append(w)
            cols.append(w ^ (N // 2))
            key = jnp.where(key == m, jnp.float32(jnp.inf), key)
    else:
        for _ in range(min(2 * B, N)):
            m = jnp.min(key, axis=1, keepdims=True)
            cols.append(jax.lax.bitcast_convert_type(m, jnp.int32) & 0x1FFF)
            key = jnp.where(key == m, jnp.float32(jnp.inf), key)
    out_ref[...] = jnp.concatenate(cols, axis=1)


def _topk_pallas(tf, T, B):
    N = tf.shape[0]
    R = min(N, 256)
    half = N >= 1024
    CC = min(2 * B, N)
    t_col = tf.reshape(N, 1)
    t_row = tf.reshape(1, N)
    return pl.pallas_call(
        functools.partial(_topk_body, T=T, B=B, R=R, N=N, half=half),
        grid=(N // R,),
        in_specs=[
            pl.BlockSpec((R, 1), lambda i: (i, 0)),
            pl.BlockSpec((1, N), lambda i: (0, 0)),
        ],
        out_specs=pl.BlockSpec((R, CC), lambda i: (i, 0)),
        out_shape=jax.ShapeDtypeStruct((N, CC), jnp.int32),
        interpret=_INTERPRET,
    )(t_col, t_row)


def _gather_dot_sc(z1f, z2f, cand, tf, T):
    N, D = z1f.shape
    CC = cand.shape[1]
    B = 8
    LOGT = T.bit_length() - 1
    info = plsc.get_sparse_core_info()
    NC, NS = info.num_cores, info.num_subcores
    NW = NC * NS
    RW = max(N // NW, 1)
    ACT = N // RW
    C = min(RW, 16)
    CHUNKS = RW // C
    PAD = 16 if CC > B else 0
    cand_flat = cand.reshape(N * CC)
    trep = (jnp.broadcast_to(tf[:, None], (N, 128)) if CC > B
            else jnp.zeros((1, 128), jnp.float32))
    mesh = plsc.VectorSubcoreMesh(core_axis_name="c", subcore_axis_name="s")

    @functools.partial(
        pl.kernel, mesh=mesh,
        out_type=jax.ShapeDtypeStruct((N, 2 * B), jnp.float32),
        scratch_types=[
            pltpu.VMEM((C * CC,), jnp.int32),
            pltpu.VMEM((C * B + PAD,), jnp.int32),
            pltpu.VMEM((C * CC, 128), jnp.float32),
            pltpu.VMEM((C, 128), jnp.float32),
            pltpu.VMEM((C, D), jnp.float32),
            pltpu.VMEM((C * B + PAD, D), jnp.float32),
            pltpu.VMEM((C * B + PAD, D), jnp.float32),
            pltpu.VMEM((C, 2 * B), jnp.float32),
            pltpu.SemaphoreType.DMA,
        ])
    def sc_kernel(z1_hbm, z2_hbm, cand_hbm, t_hbm, out_hbm,
                  cand_v, idx_v, tg_v, tro_v, own_v, g1_v, g2_v, out_v, sem):
        wid = lax.axis_index("s") * NC + lax.axis_index("c")

        @pl.when(wid < ACT)
        def _():
            lane = lax.iota(jnp.int32, 16)
            sel8 = lane < B
            perms = [((lane + s) & 15).reshape(16, 1) for s in (1, 2, 4, 8)]
            _gdims = lax.GatherDimensionNumbers(
                offset_dims=(), collapsed_slice_dims=(0,),
                start_index_map=(0,))

            def _rot(v, p):
                return lax.gather(v, p, _gdims, (1,),
                                  mode=lax.GatherScatterMode.PROMISE_IN_BOUNDS)

            def _lanesum(v):
                for p in perms:
                    v = v + _rot(v, p)
                return v

            def _lanemin(v):
                for p in perms:
                    v = jnp.minimum(v, _rot(v, p))
                return v

            def chunk_body(c, carry):
                rowbase = wid * RW + c * C
                pltpu.sync_copy(cand_hbm.at[pl.ds(rowbase * CC, C * CC)],
                                cand_v)
                if CC > B:
                    pltpu.async_copy(t_hbm.at[cand_v], tg_v, sem).wait()
                    pltpu.sync_copy(t_hbm.at[pl.ds(rowbase, C)], tro_v)
                    idx_v[pl.ds(C * B, 16)] = jnp.zeros((16,), jnp.int32)

                    def res_body(r, carry3):
                        rglob = rowbase + r
                        cd = cand_v[pl.ds(r * CC, 16)]
                        tc = jnp.zeros((16,), jnp.float32)
                        for j in range(16):
                            tc = jnp.where(lane == j,
                                           tg_v[r * CC + j, pl.ds(0, 16)], tc)
                        tr = tro_v[r, pl.ds(0, 16)]
                        dif = jnp.abs(tr - tc)
                        same = (cd >> LOGT) == (rglob >> LOGT)
                        msk = jnp.where(same, jnp.float32(1e6), dif)
                        keyv = jax.lax.bitcast_convert_type(
                            (jax.lax.bitcast_convert_type(msk, jnp.int32)
                             & ~0x1FFF) + (0x00800000 + cd), jnp.float32)
                        ids = jnp.zeros((16,), jnp.int32)
                        for k in range(B):
                            m = _lanemin(keyv)
                            ids = jnp.where(
                                lane == k,
                                jax.lax.bitcast_convert_type(m, jnp.int32)
                                & 0x1FFF, ids)
                            keyv = jnp.where(keyv == m, jnp.float32(jnp.inf),
                                             keyv)
                        idx_v[pl.ds(r * B, 16)] = ids
                        return carry3

                    lax.fori_loop(0, C, res_body, 0)
                    gidx = idx_v
                else:
                    gidx = cand_v
                pltpu.async_copy(z1_hbm.at[gidx], g1_v, sem).wait()
                pltpu.async_copy(z2_hbm.at[gidx], g2_v, sem).wait()
                pltpu.sync_copy(z1_hbm.at[pl.ds(rowbase, C)], own_v)

                def row_body(r, carry2):
                    z1r = [own_v[r, pl.ds(i * 16, 16)] for i in range(D // 16)]
                    vals = jnp.zeros((16,), jnp.float32)
                    for k in range(B):
                        acc1 = z1r[0] * g1_v[r * B + k, pl.ds(0, 16)]
                        acc2 = z1r[0] * g2_v[r * B + k, pl.ds(0, 16)]
                        for i in range(1, D // 16):
                            acc1 = acc1 + z1r[i] * g1_v[r * B + k, pl.ds(i * 16, 16)]
                            acc2 = acc2 + z1r[i] * g2_v[r * B + k, pl.ds(i * 16, 16)]
                        vals = jnp.where(lane == k, _lanesum(acc1), vals)
                        vals = jnp.where(lane == (k + B), _lanesum(acc2), vals)
                    out_v[r, :] = vals
                    return carry2

                lax.fori_loop(0, C, row_body, 0)
                pltpu.sync_copy(out_v, out_hbm.at[pl.ds(rowbase, C)])
                return carry

            lax.fori_loop(0, CHUNKS, chunk_body, 0)

    return sc_kernel(z1f, z2f, cand_flat, trep)


def _loss_body(z1_ref, z2_ref, neg_ref, out_ref):
    i = pl.program_id(0)
    z1 = z1_ref[...]
    z2 = z2_ref[...]
    neg = neg_ref[...]
    pos = jnp.sum(z1 * z2, axis=1, keepdims=True)
    m = jnp.maximum(jnp.max(neg, axis=1, keepdims=True), pos)
    s = jnp.sum(jnp.exp(neg - m), axis=1, keepdims=True) + jnp.exp(pos - m)
    lse = m + jnp.log(s)
    part = jnp.sum(lse - pos).reshape(1, 1)

    @pl.when(i == 0)
    def _():
        out_ref[...] = jnp.zeros((1, 1), jnp.float32)

    out_ref[...] += part


def _loss_pallas(z1f, z2f, neg):
    N, D = z1f.shape
    R = min(N, 256)
    return pl.pallas_call(
        _loss_body,
        grid=(N // R,),
        in_specs=[
            pl.BlockSpec((R, D), lambda i: (i, 0)),
            pl.BlockSpec((R, D), lambda i: (i, 0)),
            pl.BlockSpec((R, neg.shape[1]), lambda i: (i, 0)),
        ],
        out_specs=pl.BlockSpec((1, 1), lambda i: (0, 0)),
        out_shape=jax.ShapeDtypeStruct((1, 1), jnp.float32),
        interpret=_INTERPRET,
    )(z1f, z2f, neg)


def kernel(out1, out2, t):
    B, T, D = out1.shape
    z1, z2, tt = out1, out2, t.astype(jnp.float32)
    levels = []
    while True:
        Tl = z1.shape[1]
        N = B * Tl
        levels.append((z1.reshape(N, D), z2.reshape(N, D), tt.reshape(N), Tl))
        if Tl == 1:
            break
        T2 = Tl // 2
        tt = tt.reshape(B, T2, 2).mean(axis=2)
        z1 = z1.reshape(B, T2, 2, D).max(axis=2)
        z2 = z2.reshape(B, T2, 2, D).max(axis=2)
    idxs = [_topk_pallas(tf, Tl, B) for (_, _, tf, Tl) in levels]
    negs = [_gather_dot_sc(z1f, z2f, cand, tf, Tl)
            for (z1f, z2f, tf, Tl), cand in zip(levels, idxs)]
    total = jnp.float32(0.0)
    for (z1f, z2f, _, Tl), neg in zip(levels, negs):
        total = total + _loss_pallas(z1f, z2f, neg)[0, 0] / (B * Tl)
    return total / len(levels)


# speedup vs baseline: 2.1999x; 2.1999x over previous
import functools

import jax
import jax.numpy as jnp
from jax import lax
from jax.experimental import pallas as pl
from jax.experimental.pallas import tpu as pltpu
from jax.experimental.pallas import tpu_sc as plsc

_INTERPRET = False


def _topk_body(tc_ref, tr_ref, out_ref, *, T, B, R, N):
    i = pl.program_id(0)
    row_base = i * R
    trow = tc_ref[...]
    tall = tr_ref[...]
    diff = jnp.abs(trow - tall)
    col_iota = jax.lax.broadcasted_iota(jnp.int32, (R, N), 1)
    row_iota = jax.lax.broadcasted_iota(jnp.int32, (R, N), 0) + row_base
    same = (row_iota // T) == (col_iota // T)
    masked = jnp.where(same, jnp.float32(1e6), diff)
    key = jax.lax.bitcast_convert_type(
        (jax.lax.bitcast_convert_type(masked, jnp.int32) & ~0x1FFF)
        + (0x00800000 + col_iota),
        jnp.float32)
    cols = []
    for _ in range(B):
        m = jnp.min(key, axis=1, keepdims=True)
        cols.append(jax.lax.bitcast_convert_type(m, jnp.int32) & 0x1FFF)
        key = jnp.where(key == m, jnp.float32(jnp.inf), key)
    out_ref[...] = jnp.concatenate(cols, axis=1)


def _topk_pallas(tf, T, B):
    N = tf.shape[0]
    R = min(N, 256)
    t_col = tf.reshape(N, 1)
    t_row = tf.reshape(1, N)
    return pl.pallas_call(
        functools.partial(_topk_body, T=T, B=B, R=R, N=N),
        grid=(N // R,),
        in_specs=[
            pl.BlockSpec((R, 1), lambda i: (i, 0)),
            pl.BlockSpec((1, N), lambda i: (0, 0)),
        ],
        out_specs=pl.BlockSpec((R, B), lambda i: (i, 0)),
        out_shape=jax.ShapeDtypeStruct((N, B), jnp.int32),
        interpret=_INTERPRET,
    )(t_col, t_row)


def _gather_dot_sc(z1f, z2f, idx):
    N, D = z1f.shape
    B = idx.shape[1]
    info = plsc.get_sparse_core_info()
    NC, NS = info.num_cores, info.num_subcores
    NW = NC * NS
    RW = max(N // NW, 1)
    ACT = N // RW
    C = min(RW, 32)
    CHUNKS = RW // C
    idx_flat = idx.reshape(N * B)
    mesh = plsc.VectorSubcoreMesh(core_axis_name="c", subcore_axis_name="s")

    @functools.partial(
        pl.kernel, mesh=mesh,
        out_type=jax.ShapeDtypeStruct((N, 2 * B), jnp.float32),
        scratch_types=[
            pltpu.VMEM((C * B,), jnp.int32),
            pltpu.VMEM((C, D), jnp.float32),
            pltpu.VMEM((C * B, D), jnp.float32),
            pltpu.VMEM((C * B, D), jnp.float32),
            pltpu.VMEM((C, 2 * B), jnp.float32),
            pltpu.SemaphoreType.DMA,
        ])
    def sc_kernel(z1_hbm, z2_hbm, idx_hbm, out_hbm,
                  idx_v, own_v, g1_v, g2_v, out_v, sem):
        wid = lax.axis_index("s") * NC + lax.axis_index("c")

        @pl.when(wid < ACT)
        def _():
            def chunk_body(c, carry):
                rowbase = wid * RW + c * C
                pltpu.sync_copy(idx_hbm.at[pl.ds(rowbase * B, C * B)], idx_v)
                pltpu.async_copy(z1_hbm.at[idx_v], g1_v, sem).wait()
                pltpu.async_copy(z2_hbm.at[idx_v], g2_v, sem).wait()
                pltpu.sync_copy(z1_hbm.at[pl.ds(rowbase, C)], own_v)
                lane = lax.iota(jnp.int32, 16)
                perms = [((lane + s) & 15).reshape(16, 1) for s in (1, 2, 4, 8)]
                _gdims = lax.GatherDimensionNumbers(
                    offset_dims=(), collapsed_slice_dims=(0,),
                    start_index_map=(0,))

                def _lanesum(v):
                    for p in perms:
                        v = v + lax.gather(
                            v, p, _gdims, (1,),
                            mode=lax.GatherScatterMode.PROMISE_IN_BOUNDS)
                    return v

                def row_body(r, carry2):
                    z1r = [own_v[r, pl.ds(i * 16, 16)] for i in range(D // 16)]
                    vals = jnp.zeros((16,), jnp.float32)
                    for k in range(B):
                        acc1 = z1r[0] * g1_v[r * B + k, pl.ds(0, 16)]
                        acc2 = z1r[0] * g2_v[r * B + k, pl.ds(0, 16)]
                        for i in range(1, D // 16):
                            acc1 = acc1 + z1r[i] * g1_v[r * B + k, pl.ds(i * 16, 16)]
                            acc2 = acc2 + z1r[i] * g2_v[r * B + k, pl.ds(i * 16, 16)]
                        vals = jnp.where(lane == k, _lanesum(acc1), vals)
                        vals = jnp.where(lane == (k + B), _lanesum(acc2), vals)
                    out_v[r, :] = vals
                    return carry2

                lax.fori_loop(0, C, row_body, 0)
                pltpu.sync_copy(out_v, out_hbm.at[pl.ds(rowbase, C)])
                return carry

            lax.fori_loop(0, CHUNKS, chunk_body, 0)

    return sc_kernel(z1f, z2f, idx_flat)


def _loss_body(z1_ref, z2_ref, neg_ref, out_ref):
    i = pl.program_id(0)
    z1 = z1_ref[...]
    z2 = z2_ref[...]
    neg = neg_ref[...]
    pos = jnp.sum(z1 * z2, axis=1, keepdims=True)
    m = jnp.maximum(jnp.max(neg, axis=1, keepdims=True), pos)
    s = jnp.sum(jnp.exp(neg - m), axis=1, keepdims=True) + jnp.exp(pos - m)
    lse = m + jnp.log(s)
    part = jnp.sum(lse - pos).reshape(1, 1)

    @pl.when(i == 0)
    def _():
        out_ref[...] = jnp.zeros((1, 1), jnp.float32)

    out_ref[...] += part


def _loss_pallas(z1f, z2f, neg):
    N, D = z1f.shape
    R = min(N, 256)
    return pl.pallas_call(
        _loss_body,
        grid=(N // R,),
        in_specs=[
            pl.BlockSpec((R, D), lambda i: (i, 0)),
            pl.BlockSpec((R, D), lambda i: (i, 0)),
            pl.BlockSpec((R, neg.shape[1]), lambda i: (i, 0)),
        ],
        out_specs=pl.BlockSpec((1, 1), lambda i: (0, 0)),
        out_shape=jax.ShapeDtypeStruct((1, 1), jnp.float32),
        interpret=_INTERPRET,
    )(z1f, z2f, neg)


def kernel(out1, out2, t):
    B, T, D = out1.shape
    z1, z2, tt = out1, out2, t.astype(jnp.float32)
    levels = []
    while True:
        Tl = z1.shape[1]
        N = B * Tl
        levels.append((z1.reshape(N, D), z2.reshape(N, D), tt.reshape(N), Tl))
        if Tl == 1:
            break
        T2 = Tl // 2
        tt = tt.reshape(B, T2, 2).mean(axis=2)
        z1 = z1.reshape(B, T2, 2, D).max(axis=2)
        z2 = z2.reshape(B, T2, 2, D).max(axis=2)
    idxs = [_topk_pallas(tf, Tl, B) for (_, _, tf, Tl) in levels]
    negs = [_gather_dot_sc(z1f, z2f, idx)
            for (z1f, z2f, _, _), idx in zip(levels, idxs)]
    total = jnp.float32(0.0)
    for (z1f, z2f, _, Tl), neg in zip(levels, negs):
        total = total + _loss_pallas(z1f, z2f, neg)[0, 0] / (B * Tl)
    return total / len(levels)


# speedup vs baseline: 2.2081x; 1.0037x over previous
import functools

import jax
import jax.numpy as jnp
from jax import lax
from jax.experimental import pallas as pl
from jax.experimental.pallas import tpu as pltpu
from jax.experimental.pallas import tpu_sc as plsc

_INTERPRET = False


def _topk_body(tc_ref, tr_ref, out_ref, *, T, B, R, N):
    i = pl.program_id(0)
    row_base = i * R
    trow = tc_ref[...]
    tall = tr_ref[...]
    diff = jnp.abs(trow - tall)
    col_iota = jax.lax.broadcasted_iota(jnp.int32, (R, N), 1)
    row_iota = jax.lax.broadcasted_iota(jnp.int32, (R, N), 0) + row_base
    same = (row_iota // T) == (col_iota // T)
    masked = jnp.where(same, jnp.float32(1e6), diff)
    key = jax.lax.bitcast_convert_type(
        (jax.lax.bitcast_convert_type(masked, jnp.int32) & ~0x1FFF)
        + (0x00800000 + col_iota),
        jnp.float32)
    cols = []
    m = jnp.min(key, axis=1, keepdims=True)
    cols.append(jax.lax.bitcast_convert_type(m, jnp.int32) & 0x1FFF)
    for _ in range(B - 1):
        m = jnp.min(jnp.where(key > m, key, jnp.float32(jnp.inf)),
                    axis=1, keepdims=True)
        cols.append(jax.lax.bitcast_convert_type(m, jnp.int32) & 0x1FFF)
    out_ref[...] = jnp.concatenate(cols, axis=1)


def _topk_pallas(tf, T, B):
    N = tf.shape[0]
    R = min(N, 256)
    t_col = tf.reshape(N, 1)
    t_row = tf.reshape(1, N)
    return pl.pallas_call(
        functools.partial(_topk_body, T=T, B=B, R=R, N=N),
        grid=(N // R,),
        in_specs=[
            pl.BlockSpec((R, 1), lambda i: (i, 0)),
            pl.BlockSpec((1, N), lambda i: (0, 0)),
        ],
        out_specs=pl.BlockSpec((R, B), lambda i: (i, 0)),
        out_shape=jax.ShapeDtypeStruct((N, B), jnp.int32),
        interpret=_INTERPRET,
    )(t_col, t_row)


def _gather_dot_sc(z1f, z2f, idx):
    N, D = z1f.shape
    B = idx.shape[1]
    info = plsc.get_sparse_core_info()
    NC, NS = info.num_cores, info.num_subcores
    NW = NC * NS
    RW = max(N // NW, 1)
    ACT = N // RW
    C = min(RW, 32)
    CHUNKS = RW // C
    idx_flat = idx.reshape(N * B)
    mesh = plsc.VectorSubcoreMesh(core_axis_name="c", subcore_axis_name="s")

    @functools.partial(
        pl.kernel, mesh=mesh,
        out_type=jax.ShapeDtypeStruct((N, 2 * B), jnp.float32),
        scratch_types=[
            pltpu.VMEM((C * B,), jnp.int32),
            pltpu.VMEM((C, D), jnp.float32),
            pltpu.VMEM((C * B, D), jnp.float32),
            pltpu.VMEM((C * B, D), jnp.float32),
            pltpu.VMEM((C, 2 * B), jnp.float32),
            pltpu.SemaphoreType.DMA,
        ])
    def sc_kernel(z1_hbm, z2_hbm, idx_hbm, out_hbm,
                  idx_v, own_v, g1_v, g2_v, out_v, sem):
        wid = lax.axis_index("s") * NC + lax.axis_index("c")

        @pl.when(wid < ACT)
        def _():
            def chunk_body(c, carry):
                rowbase = wid * RW + c * C
                pltpu.sync_copy(idx_hbm.at[pl.ds(rowbase * B, C * B)], idx_v)
                pltpu.async_copy(z1_hbm.at[idx_v], g1_v, sem).wait()
                pltpu.async_copy(z2_hbm.at[idx_v], g2_v, sem).wait()
                pltpu.sync_copy(z1_hbm.at[pl.ds(rowbase, C)], own_v)
                lane = lax.iota(jnp.int32, 16)
                perms = [((lane + s) & 15).reshape(16, 1) for s in (1, 2, 4, 8)]
                _gdims = lax.GatherDimensionNumbers(
                    offset_dims=(), collapsed_slice_dims=(0,),
                    start_index_map=(0,))

                def _lanesum(v):
                    for p in perms:
                        v = v + lax.gather(
                            v, p, _gdims, (1,),
                            mode=lax.GatherScatterMode.PROMISE_IN_BOUNDS)
                    return v

                def row_body(r, carry2):
                    z1r = [own_v[r, pl.ds(i * 16, 16)] for i in range(D // 16)]
                    vals = jnp.zeros((16,), jnp.float32)
                    for k in range(B):
                        acc1 = z1r[0] * g1_v[r * B + k, pl.ds(0, 16)]
                        acc2 = z1r[0] * g2_v[r * B + k, pl.ds(0, 16)]
                        for i in range(1, D // 16):
                            acc1 = acc1 + z1r[i] * g1_v[r * B + k, pl.ds(i * 16, 16)]
                            acc2 = acc2 + z1r[i] * g2_v[r * B + k, pl.ds(i * 16, 16)]
                        vals = jnp.where(lane == k, _lanesum(acc1), vals)
                        vals = jnp.where(lane == (k + B), _lanesum(acc2), vals)
                    out_v[r, :] = vals
                    return carry2

                lax.fori_loop(0, C, row_body, 0)
                pltpu.sync_copy(out_v, out_hbm.at[pl.ds(rowbase, C)])
                return carry

            lax.fori_loop(0, CHUNKS, chunk_body, 0)

    return sc_kernel(z1f, z2f, idx_flat)


def _loss_body(z1_ref, z2_ref, neg_ref, out_ref):
    i = pl.program_id(0)
    z1 = z1_ref[...]
    z2 = z2_ref[...]
    neg = neg_ref[...]
    pos = jnp.sum(z1 * z2, axis=1, keepdims=True)
    m = jnp.maximum(jnp.max(neg, axis=1, keepdims=True), pos)
    s = jnp.sum(jnp.exp(neg - m), axis=1, keepdims=True) + jnp.exp(pos - m)
    lse = m + jnp.log(s)
    part = jnp.sum(lse - pos).reshape(1, 1)

    @pl.when(i == 0)
    def _():
        out_ref[...] = jnp.zeros((1, 1), jnp.float32)

    out_ref[...] += part


def _loss_pallas(z1f, z2f, neg):
    N, D = z1f.shape
    R = min(N, 256)
    return pl.pallas_call(
        _loss_body,
        grid=(N // R,),
        in_specs=[
            pl.BlockSpec((R, D), lambda i: (i, 0)),
            pl.BlockSpec((R, D), lambda i: (i, 0)),
            pl.BlockSpec((R, neg.shape[1]), lambda i: (i, 0)),
        ],
        out_specs=pl.BlockSpec((1, 1), lambda i: (0, 0)),
        out_shape=jax.ShapeDtypeStruct((1, 1), jnp.float32),
        interpret=_INTERPRET,
    )(z1f, z2f, neg)


def kernel(out1, out2, t):
    B, T, D = out1.shape
    z1, z2, tt = out1, out2, t.astype(jnp.float32)
    levels = []
    while True:
        Tl = z1.shape[1]
        N = B * Tl
        levels.append((z1.reshape(N, D), z2.reshape(N, D), tt.reshape(N), Tl))
        if Tl == 1:
            break
        T2 = Tl // 2
        tt = tt.reshape(B, T2, 2).mean(axis=2)
        z1 = z1.reshape(B, T2, 2, D).max(axis=2)
        z2 = z2.reshape(B, T2, 2, D).max(axis=2)
    idxs = [_topk_pallas(tf, Tl, B) for (_, _, tf, Tl) in levels]
    negs = [_gather_dot_sc(z1f, z2f, idx)
            for (z1f, z2f, _, _), idx in zip(levels, idxs)]
    total = jnp.float32(0.0)
    for (z1f, z2f, _, Tl), neg in zip(levels, negs):
        total = total + _loss_pallas(z1f, z2f, neg)[0, 0] / (B * Tl)
    return total / len(levels)
